# baseline (speedup 1.0000x reference)
import functools

import jax
import jax.numpy as jnp
from jax import lax
from jax.experimental import pallas as pl
from jax.experimental.pallas import tpu as pltpu
from jax.experimental.pallas import tpu_sc as plsc

N = 10000
D = 128
E = 320000
NC = 2
NS = 16
NW = NC * NS
EPW = E // NW
CHUNK = 128
NFULL = EPW // CHUNK
TAIL = EPW - NFULL * CHUNK
RPT = N // NS
ZROWS = 25
WRows = 640
WLAST = N - 15 * WRows

_mesh = plsc.VectorSubcoreMesh(core_axis_name="c", subcore_axis_name="s")

_f32 = jnp.float32
_i32 = jnp.int32


CHUNK2 = 80
NCH = 128
SLOTS = NCH * CHUNK2
NGRP = NCH // 4
ACC_ROWS = N + 16
CB = CHUNK2 * D * 4


@functools.partial(
    pl.kernel,
    mesh=_mesh,
    out_type=jax.ShapeDtypeStruct((NC * N, D), _f32),
    scratch_types=[
        pltpu.VMEM((CHUNK2,), _i32),
        pltpu.VMEM((CHUNK2,), _i32),
        pltpu.VMEM((CHUNK2,), _i32),
        pltpu.VMEM((CHUNK2,), _i32),
        pltpu.VMEM((CHUNK2,), _i32),
        pltpu.VMEM((CHUNK2,), _i32),
        pltpu.VMEM((CHUNK2,), _i32),
        pltpu.VMEM((CHUNK2,), _i32),
        pltpu.VMEM((CHUNK2, D), _f32),
        pltpu.VMEM((CHUNK2, D), _f32),
        pltpu.VMEM((CHUNK2, D), _f32),
        pltpu.VMEM((CHUNK2, D), _f32),
        pltpu.VMEM((ZROWS, D), _f32),
        pltpu.VMEM_SHARED((ACC_ROWS, D), _f32),
        pltpu.SemaphoreType.DMA,
        pltpu.SemaphoreType.DMA,
        pltpu.SemaphoreType.DMA,
    ],
)
def _prop_sc(u_hbm, col_hbm, row_hbm, out_hbm,
             ca0, ca1, ra0, ra1, cb0, cb1, rb0, rb1,
             b0, b1, b2, b3, zbuf, acc, gsem, ssem, isem):
    c = lax.axis_index("c")
    s = lax.axis_index("s")
    w = s * NC + c
    base = w * SLOTS
    zeros16 = jnp.zeros((16,), _f32)

    def zfill(i, _):
        for j in range(D // 16):
            zbuf[i, pl.ds(j * 16, 16)] = zeros16
        return 0
    lax.fori_loop(0, ZROWS, zfill, 0)

    def zero(i, _):
        pltpu.sync_copy(zbuf, acc.at[pl.ds(s * RPT + i * ZROWS, ZROWS)])
        return 0
    lax.fori_loop(0, RPT // ZROWS, zero, 0)

    pltpu.sync_copy(col_hbm.at[pl.ds(base, CHUNK2)], ca0)
    pltpu.sync_copy(col_hbm.at[pl.ds(base + CHUNK2, CHUNK2)], ca1)
    pltpu.sync_copy(row_hbm.at[pl.ds(base, CHUNK2)], ra0)
    pltpu.sync_copy(row_hbm.at[pl.ds(base + CHUNK2, CHUNK2)], ra1)
    plsc.subcore_barrier()

    def _drain(sem, dst, count):
        for _ in range(count):
            pltpu.make_async_copy(u_hbm.at[pl.ds(0, dst.shape[0])]
                                  if len(dst.shape) == 2 else
                                  col_hbm.at[pl.ds(0, dst.shape[0])],
                                  dst, sem).wait()

    def body(g, _):
        off = base + g * 4 * CHUNK2

        @pl.when(g > 0)
        def _():
            _drain(isem, ca0, 4)
        ga0 = pltpu.async_copy(u_hbm.at[ca0], b0, gsem)
        ga1 = pltpu.async_copy(u_hbm.at[ca1], b1, gsem)

        @pl.when(g > 0)
        def _():
            _drain(ssem, b2, 2)
        pltpu.async_copy(col_hbm.at[pl.ds(off + 2 * CHUNK2, CHUNK2)],
                         cb0, isem)
        pltpu.async_copy(col_hbm.at[pl.ds(off + 3 * CHUNK2, CHUNK2)],
                         cb1, isem)
        pltpu.async_copy(row_hbm.at[pl.ds(off + 2 * CHUNK2, CHUNK2)],
                         rb0, isem)
        pltpu.async_copy(row_hbm.at[pl.ds(off + 3 * CHUNK2, CHUNK2)],
                         rb1, isem)
        ga0.wait()
        ga1.wait()
        pltpu.async_copy(b0, acc.at[ra0], ssem, add=True)
        pltpu.async_copy(b1, acc.at[ra1], ssem, add=True)
        _drain(isem, cb0, 4)
        gb0 = pltpu.async_copy(u_hbm.at[cb0], b2, gsem)
        gb1 = pltpu.async_copy(u_hbm.at[cb1], b3, gsem)
        _drain(ssem, b0, 2)

        @pl.when(g < NGRP - 1)
        def _():
            offn = off + 4 * CHUNK2
            pltpu.async_copy(col_hbm.at[pl.ds(offn, CHUNK2)], ca0, isem)
            pltpu.async_copy(col_hbm.at[pl.ds(offn + CHUNK2, CHUNK2)],
                             ca1, isem)
            pltpu.async_copy(row_hbm.at[pl.ds(offn, CHUNK2)], ra0, isem)
            pltpu.async_copy(row_hbm.at[pl.ds(offn + CHUNK2, CHUNK2)],
                             ra1, isem)
        gb0.wait()
        gb1.wait()
        pltpu.async_copy(b2, acc.at[rb0], ssem, add=True)
        pltpu.async_copy(b3, acc.at[rb1], ssem, add=True)
        return 0
    lax.fori_loop(0, NGRP, body, 0)
    _drain(ssem, b2, 2)
    plsc.subcore_barrier()

    @pl.when(s < NS - 1)
    def _():
        pltpu.sync_copy(acc.at[pl.ds(s * WRows, WRows)],
                        out_hbm.at[pl.ds(c * N + s * WRows, WRows)])

    @pl.when(s == NS - 1)
    def _():
        pltpu.sync_copy(acc.at[pl.ds(15 * WRows, WLAST)],
                        out_hbm.at[pl.ds(c * N + 15 * WRows, WLAST)])


BM = 2000
GRID = N // BM

_sd = jax.ShapeDtypeStruct


def _dot(a, b):
    return jax.lax.dot_general(a, b, (((1,), (0,)), ((), ())),
                               precision=jax.lax.Precision.HIGHEST,
                               preferred_element_type=_f32)


def _stats_body(x_ref, m_ref, r_ref):
    x = x_ref[...]
    m = jnp.mean(x, axis=0, keepdims=True)
    m_ref[...] = m
    v = jnp.mean((x - m) * (x - m), axis=0, keepdims=True)
    r_ref[...] = jax.lax.rsqrt(v + 1e-5)


_bn_stats = pl.pallas_call(
    _stats_body,
    out_shape=(_sd((1, D), _f32), _sd((1, D), _f32)))


def _blk(shape):
    return pl.BlockSpec(shape, lambda i: (i, 0))


def _blk2(shape):
    return pl.BlockSpec(shape, lambda i: (i + GRID, 0))


def _full(shape):
    return pl.BlockSpec(shape, lambda i: (0, 0))


def _ones_body(m_ref, o_ref):
    o_ref[...] = jnp.broadcast_to(m_ref[...] * 0.0 + 1.0, (BM, D))


_ones_tc = pl.pallas_call(
    _ones_body,
    grid=(GRID,),
    in_specs=[_full((1, D))],
    out_specs=_blk((BM, D)),
    out_shape=_sd((N, D), _f32))


def _scale_body(x_ref, m_ref, r_ref, g_ref, be_ref, dga_ref, dgb_ref,
                xb_ref, dinv_ref, u1_ref):
    xb = ((x_ref[...] - m_ref[...]) * r_ref[...] * g_ref[...]
          + be_ref[...])
    xb_ref[...] = xb
    deg = dga_ref[:, 0:1] + dgb_ref[:, 0:1]
    dinv = jnp.where(deg > 0.0,
                     jax.lax.rsqrt(jnp.maximum(deg, 1.0)), 0.0)
    dinv_ref[...] = dinv
    u1_ref[...] = xb * dinv


_scale = pl.pallas_call(
    _scale_body,
    grid=(GRID,),
    in_specs=[_blk((BM, D)), _full((1, D)), _full((1, D)),
              _full((1, D)), _full((1, D)),
              _blk((BM, D)), _blk2((BM, D))],
    out_specs=(_blk((BM, D)), _blk((BM, 1)), _blk((BM, D))),
    out_shape=(_sd((N, D), _f32), _sd((N, 1), _f32), _sd((N, D), _f32)))


def _t3_body(sa_ref, sb_ref, dinv_ref, xb_ref, w0_ref, w1_ref, b_ref,
             g2_ref, acc1_ref):
    dinv = dinv_ref[...]
    tx1 = -dinv * (sa_ref[...] + sb_ref[...])
    g2_ref[...] = dinv * tx1
    acc1_ref[...] = (_dot(xb_ref[...], w0_ref[...])
                     + _dot(tx1, w1_ref[...]) + b_ref[...])


_t3 = pl.pallas_call(
    _t3_body,
    grid=(GRID,),
    in_specs=[_blk((BM, D)), _blk2((BM, D)), _blk((BM, 1)), _blk((BM, D)),
              _full((D, D)), _full((D, D)), _full((1, D))],
    out_specs=(_blk((BM, D)), _blk((BM, D))),
    out_shape=(_sd((N, D), _f32), _sd((N, D), _f32)))


def _t4a_body(sa_ref, sb_ref, dinv_ref, xb_ref, acc1_ref, w2_ref,
              h_ref):
    tx2 = (-2.0 * dinv_ref[...] * (sa_ref[...] + sb_ref[...])
           - xb_ref[...])
    h_ref[...] = jnp.maximum(acc1_ref[...] + _dot(tx2, w2_ref[...]), 0.0)


_t4a = pl.pallas_call(
    _t4a_body,
    grid=(GRID,),
    in_specs=[_blk((BM, D)), _blk2((BM, D)), _blk((BM, 1)), _blk((BM, D)),
              _blk((BM, D)), _full((D, D))],
    out_specs=_blk((BM, D)),
    out_shape=_sd((N, D), _f32))


def _t4b_body(h_ref, m_ref, r_ref, g_ref, be_ref, dinv_ref,
              hb_ref, g3_ref):
    hb = ((h_ref[...] - m_ref[...]) * r_ref[...] * g_ref[...]
          + be_ref[...])
    hb_ref[...] = hb
    g3_ref[...] = dinv_ref[...] * hb


_t4b = pl.pallas_call(
    _t4b_body,
    grid=(GRID,),
    in_specs=[_blk((BM, D)), _full((1, D)), _full((1, D)),
              _full((1, D)), _full((1, D)), _blk((BM, 1))],
    out_specs=(_blk((BM, D)), _blk((BM, D))),
    out_shape=(_sd((N, D), _f32), _sd((N, D), _f32)))


def _t5_body(sa_ref, sb_ref, dinv_ref, hb_ref, w0_ref, w1_ref, b_ref,
             acc2_ref, g4_ref):
    dinv = dinv_ref[...]
    ty1 = -dinv * (sa_ref[...] + sb_ref[...])
    g4_ref[...] = dinv * ty1
    acc2_ref[...] = (_dot(hb_ref[...], w0_ref[...])
                     + _dot(ty1, w1_ref[...]) + b_ref[...])


_t5 = pl.pallas_call(
    _t5_body,
    grid=(GRID,),
    in_specs=[_blk((BM, D)), _blk2((BM, D)), _blk((BM, 1)), _blk((BM, D)),
              _full((D, D)), _full((D, D)), _full((1, D))],
    out_specs=(_blk((BM, D)), _blk((BM, D))),
    out_shape=(_sd((N, D), _f32), _sd((N, D), _f32)))


def _t6_body(sa_ref, sb_ref, dinv_ref, hb_ref, acc2_ref, xb_ref, w2_ref,
             o_ref):
    ty2 = (-2.0 * dinv_ref[...] * (sa_ref[...] + sb_ref[...])
           - hb_ref[...])
    o_ref[...] = jnp.maximum(
        xb_ref[...] + acc2_ref[...] + _dot(ty2, w2_ref[...]), 0.0)


_t6 = pl.pallas_call(
    _t6_body,
    grid=(GRID,),
    in_specs=[_blk((BM, D)), _blk2((BM, D)), _blk((BM, 1)), _blk((BM, D)),
              _blk((BM, D)), _blk((BM, D)), _full((D, D))],
    out_specs=_blk((BM, D)),
    out_shape=_sd((N, D), _f32))


def kernel(x, edge_index, bn1_gamma, bn1_beta, W1, b1,
           bn2_gamma, bn2_beta, W2, b2):
    row = edge_index[0].astype(_i32)
    col = edge_index[1].astype(_i32)
    pad = SLOTS - EPW
    col_p = jnp.pad(col.reshape(NW, EPW), ((0, 0), (0, pad))
                    ).reshape(NW * SLOTS)
    row_p = jnp.pad(row.reshape(NW, EPW), ((0, 0), (0, pad)),
                    constant_values=N).reshape(NW * SLOTS)
    g1 = bn1_gamma.reshape(1, D)
    be1 = bn1_beta.reshape(1, D)
    g2r = bn2_gamma.reshape(1, D)
    be2 = bn2_beta.reshape(1, D)
    b1r = b1.reshape(1, D)
    b2r = b2.reshape(1, D)

    m1, r1 = _bn_stats(x)
    degp = _prop_sc(_ones_tc(m1), col_p, row_p)
    xb, dinv, u1 = _scale(x, m1, r1, g1, be1, degp, degp)
    s1 = _prop_sc(u1, col_p, row_p)
    g2, acc1 = _t3(s1, s1, dinv, xb, W1[0], W1[1], b1r)
    s2 = _prop_sc(g2, col_p, row_p)
    h = _t4a(s2, s2, dinv, xb, acc1, W1[2])
    m2, r2 = _bn_stats(h)
    hb, g3 = _t4b(h, m2, r2, g2r, be2, dinv)
    s3 = _prop_sc(g3, col_p, row_p)
    acc2, g4 = _t5(s3, s3, dinv, hb, W2[0], W2[1], b2r)
    s4 = _prop_sc(g4, col_p, row_p)
    return _t6(s4, s4, dinv, hb, acc2, xb, W2[2])


# speedup vs baseline: 1.1552x; 1.1552x over previous
import functools

import jax
import jax.numpy as jnp
from jax import lax
from jax.experimental import pallas as pl
from jax.experimental.pallas import tpu as pltpu
from jax.experimental.pallas import tpu_sc as plsc

N = 10000
D = 128
E = 320000
NC = 2
NS = 16
NW = NC * NS
EPW = E // NW
CHUNK = 128
NFULL = EPW // CHUNK
TAIL = EPW - NFULL * CHUNK
RPT = N // NS
ZROWS = 25
WRows = 640
WLAST = N - 15 * WRows

_mesh = plsc.VectorSubcoreMesh(core_axis_name="c", subcore_axis_name="s")

_f32 = jnp.float32
_i32 = jnp.int32


CH3 = 128
NCH3 = 80
PRELOAD = 40
SLOTS = NCH3 * CH3
ACC_ROWS = N + 8


@functools.partial(
    pl.kernel,
    mesh=_mesh,
    out_type=jax.ShapeDtypeStruct((NC * N, D), _f32),
    scratch_types=[
        pltpu.VMEM((CHUNK,), _i32),
        pltpu.VMEM((TAIL,), _i32),
        pltpu.VMEM((CHUNK, D), _f32),
        pltpu.VMEM((TAIL, D), _f32),
        pltpu.VMEM((ZROWS, D), _f32),
        pltpu.VMEM_SHARED((N, D), _f32),
    ],
)
def _deg_sc(row_hbm, out_hbm, rowv, rowt, onesb, onest, zbuf, acc):
    c = lax.axis_index("c")
    s = lax.axis_index("s")
    w = s * NC + c
    ones16 = jnp.ones((16,), _f32)
    zeros16 = jnp.zeros((16,), _f32)

    def fill(i, _):
        for j in range(D // 16):
            onesb[i, pl.ds(j * 16, 16)] = ones16
            zbuf[lax.rem(i, ZROWS), pl.ds(j * 16, 16)] = zeros16
        return 0
    lax.fori_loop(0, CHUNK, fill, 0)

    def fill_t(i, _):
        for j in range(D // 16):
            onest[i, pl.ds(j * 16, 16)] = ones16
        return 0
    lax.fori_loop(0, TAIL, fill_t, 0)

    def zero(i, _):
        pltpu.sync_copy(zbuf, acc.at[pl.ds(s * RPT + i * ZROWS, ZROWS)])
        return 0
    lax.fori_loop(0, RPT // ZROWS, zero, 0)
    plsc.subcore_barrier()

    base = w * EPW

    def step(j, _):
        pltpu.sync_copy(row_hbm.at[pl.ds(base + j * CHUNK, CHUNK)], rowv)
        pltpu.sync_copy(onesb, acc.at[rowv], add=True)
        return 0
    lax.fori_loop(0, NFULL, step, 0)
    pltpu.sync_copy(row_hbm.at[pl.ds(base + NFULL * CHUNK, TAIL)], rowt)
    pltpu.sync_copy(onest, acc.at[rowt], add=True)
    plsc.subcore_barrier()

    @pl.when(s < NS - 1)
    def _():
        pltpu.sync_copy(acc.at[pl.ds(s * WRows, WRows)],
                        out_hbm.at[pl.ds(c * N + s * WRows, WRows)])

    @pl.when(s == NS - 1)
    def _():
        pltpu.sync_copy(acc.at[pl.ds(15 * WRows, WLAST)],
                        out_hbm.at[pl.ds(c * N + 15 * WRows, WLAST)])


@functools.partial(
    pl.kernel,
    mesh=_mesh,
    out_type=jax.ShapeDtypeStruct((NC * N, D), _f32),
    scratch_types=[
        pltpu.VMEM((PRELOAD, CH3), _i32),
        pltpu.VMEM((PRELOAD, CH3), _i32),
        pltpu.VMEM((CH3, D), _f32),
        pltpu.VMEM((CH3, D), _f32),
        pltpu.VMEM_SHARED((ACC_ROWS, D), _f32),
        pltpu.SemaphoreType.DMA,
        pltpu.SemaphoreType.DMA,
        pltpu.SemaphoreType.DMA,
        pltpu.SemaphoreType.DMA,
    ],
)
def _prop_sc(u_hbm, col_hbm, row_hbm, out_hbm,
             colb, rowb, b0, b1, acc, gsem0, gsem1, ssem0, ssem1):
    c = lax.axis_index("c")
    s = lax.axis_index("s")
    w = s * NC + c
    zeros16 = jnp.zeros((16,), _f32)

    def zfill(i, _):
        for j in range(D // 16):
            b0[i, pl.ds(j * 16, 16)] = zeros16
        return 0
    lax.fori_loop(0, CH3, zfill, 0)
    for k in range(RPT // CH3):
        pltpu.sync_copy(b0, acc.at[pl.ds(s * RPT + k * CH3, CH3)])
    _rem = RPT - (RPT // CH3) * CH3
    pltpu.sync_copy(b0.at[pl.ds(0, _rem)],
                    acc.at[pl.ds(s * RPT + (RPT // CH3) * CH3, _rem)])
    plsc.subcore_barrier()

    def _gwait(sem, buf):
        pltpu.make_async_copy(u_hbm.at[pl.ds(0, CH3)], buf, sem).wait()

    def phase(pbase):
        pltpu.sync_copy(col_hbm.at[pl.ds(w * NCH3 + pbase, PRELOAD)], colb)
        pltpu.sync_copy(row_hbm.at[pl.ds(w * NCH3 + pbase, PRELOAD)], rowb)
        pltpu.async_copy(u_hbm.at[colb.at[0]], b0, gsem0)

        def body(g, _):
            j0 = 2 * g
            _gwait(gsem0, b0)

            @pl.when(g > 0)
            def _():
                _gwait(ssem1, b1)
            pltpu.async_copy(u_hbm.at[colb.at[j0 + 1]], b1, gsem1)
            pltpu.async_copy(b0, acc.at[rowb.at[j0]], ssem0, add=True)
            _gwait(gsem1, b1)
            _gwait(ssem0, b0)

            @pl.when(g < PRELOAD // 2 - 1)
            def _():
                pltpu.async_copy(u_hbm.at[colb.at[j0 + 2]], b0, gsem0)
            pltpu.async_copy(b1, acc.at[rowb.at[j0 + 1]], ssem1, add=True)
            return 0
        lax.fori_loop(0, PRELOAD // 2, body, 0)
        _gwait(ssem1, b1)

    phase(0)
    phase(PRELOAD)
    plsc.subcore_barrier()

    @pl.when(s < NS - 1)
    def _():
        pltpu.sync_copy(acc.at[pl.ds(s * WRows, WRows)],
                        out_hbm.at[pl.ds(c * N + s * WRows, WRows)])

    @pl.when(s == NS - 1)
    def _():
        pltpu.sync_copy(acc.at[pl.ds(15 * WRows, WLAST)],
                        out_hbm.at[pl.ds(c * N + 15 * WRows, WLAST)])


BM = 2000
GRID = N // BM

_sd = jax.ShapeDtypeStruct


def _dot(a, b):
    return jax.lax.dot_general(a, b, (((1,), (0,)), ((), ())),
                               precision=jax.lax.Precision.HIGHEST,
                               preferred_element_type=_f32)


def _stats_body(x_ref, m_ref, r_ref):
    x = x_ref[...]
    m = jnp.mean(x, axis=0, keepdims=True)
    m_ref[...] = m
    v = jnp.mean((x - m) * (x - m), axis=0, keepdims=True)
    r_ref[...] = jax.lax.rsqrt(v + 1e-5)


_bn_stats = pl.pallas_call(
    _stats_body,
    out_shape=(_sd((1, D), _f32), _sd((1, D), _f32)))


def _blk(shape):
    return pl.BlockSpec(shape, lambda i: (i, 0))


def _blk2(shape):
    return pl.BlockSpec(shape, lambda i: (i + GRID, 0))


def _full(shape):
    return pl.BlockSpec(shape, lambda i: (0, 0))


def _ones_body(m_ref, o_ref):
    o_ref[...] = jnp.broadcast_to(m_ref[...] * 0.0 + 1.0, (BM, D))


_ones_tc = pl.pallas_call(
    _ones_body,
    grid=(GRID,),
    in_specs=[_full((1, D))],
    out_specs=_blk((BM, D)),
    out_shape=_sd((N, D), _f32))


def _scale_body(x_ref, m_ref, r_ref, g_ref, be_ref, dga_ref, dgb_ref,
                xb_ref, dinv_ref, u1_ref):
    xb = ((x_ref[...] - m_ref[...]) * r_ref[...] * g_ref[...]
          + be_ref[...])
    xb_ref[...] = xb
    deg = dga_ref[:, 0:1] + dgb_ref[:, 0:1]
    dinv = jnp.where(deg > 0.0,
                     jax.lax.rsqrt(jnp.maximum(deg, 1.0)), 0.0)
    dinv_ref[...] = dinv
    u1_ref[...] = xb * dinv


_scale = pl.pallas_call(
    _scale_body,
    grid=(GRID,),
    in_specs=[_blk((BM, D)), _full((1, D)), _full((1, D)),
              _full((1, D)), _full((1, D)),
              _blk((BM, D)), _blk2((BM, D))],
    out_specs=(_blk((BM, D)), _blk((BM, 1)), _blk((BM, D))),
    out_shape=(_sd((N, D), _f32), _sd((N, 1), _f32), _sd((N, D), _f32)))


def _t3_body(sa_ref, sb_ref, dinv_ref, xb_ref, w0_ref, w1_ref, b_ref,
             g2_ref, acc1_ref):
    dinv = dinv_ref[...]
    tx1 = -dinv * (sa_ref[...] + sb_ref[...])
    g2_ref[...] = dinv * tx1
    acc1_ref[...] = (_dot(xb_ref[...], w0_ref[...])
                     + _dot(tx1, w1_ref[...]) + b_ref[...])


_t3 = pl.pallas_call(
    _t3_body,
    grid=(GRID,),
    in_specs=[_blk((BM, D)), _blk2((BM, D)), _blk((BM, 1)), _blk((BM, D)),
              _full((D, D)), _full((D, D)), _full((1, D))],
    out_specs=(_blk((BM, D)), _blk((BM, D))),
    out_shape=(_sd((N, D), _f32), _sd((N, D), _f32)))


def _t4a_body(sa_ref, sb_ref, dinv_ref, xb_ref, acc1_ref, w2_ref,
              h_ref):
    tx2 = (-2.0 * dinv_ref[...] * (sa_ref[...] + sb_ref[...])
           - xb_ref[...])
    h_ref[...] = jnp.maximum(acc1_ref[...] + _dot(tx2, w2_ref[...]), 0.0)


_t4a = pl.pallas_call(
    _t4a_body,
    grid=(GRID,),
    in_specs=[_blk((BM, D)), _blk2((BM, D)), _blk((BM, 1)), _blk((BM, D)),
              _blk((BM, D)), _full((D, D))],
    out_specs=_blk((BM, D)),
    out_shape=_sd((N, D), _f32))


def _t4b_body(h_ref, m_ref, r_ref, g_ref, be_ref, dinv_ref,
              hb_ref, g3_ref):
    hb = ((h_ref[...] - m_ref[...]) * r_ref[...] * g_ref[...]
          + be_ref[...])
    hb_ref[...] = hb
    g3_ref[...] = dinv_ref[...] * hb


_t4b = pl.pallas_call(
    _t4b_body,
    grid=(GRID,),
    in_specs=[_blk((BM, D)), _full((1, D)), _full((1, D)),
              _full((1, D)), _full((1, D)), _blk((BM, 1))],
    out_specs=(_blk((BM, D)), _blk((BM, D))),
    out_shape=(_sd((N, D), _f32), _sd((N, D), _f32)))


def _t5_body(sa_ref, sb_ref, dinv_ref, hb_ref, w0_ref, w1_ref, b_ref,
             acc2_ref, g4_ref):
    dinv = dinv_ref[...]
    ty1 = -dinv * (sa_ref[...] + sb_ref[...])
    g4_ref[...] = dinv * ty1
    acc2_ref[...] = (_dot(hb_ref[...], w0_ref[...])
                     + _dot(ty1, w1_ref[...]) + b_ref[...])


_t5 = pl.pallas_call(
    _t5_body,
    grid=(GRID,),
    in_specs=[_blk((BM, D)), _blk2((BM, D)), _blk((BM, 1)), _blk((BM, D)),
              _full((D, D)), _full((D, D)), _full((1, D))],
    out_specs=(_blk((BM, D)), _blk((BM, D))),
    out_shape=(_sd((N, D), _f32), _sd((N, D), _f32)))


def _t6_body(sa_ref, sb_ref, dinv_ref, hb_ref, acc2_ref, xb_ref, w2_ref,
             o_ref):
    ty2 = (-2.0 * dinv_ref[...] * (sa_ref[...] + sb_ref[...])
           - hb_ref[...])
    o_ref[...] = jnp.maximum(
        xb_ref[...] + acc2_ref[...] + _dot(ty2, w2_ref[...]), 0.0)


_t6 = pl.pallas_call(
    _t6_body,
    grid=(GRID,),
    in_specs=[_blk((BM, D)), _blk2((BM, D)), _blk((BM, 1)), _blk((BM, D)),
              _blk((BM, D)), _blk((BM, D)), _full((D, D))],
    out_specs=_blk((BM, D)),
    out_shape=_sd((N, D), _f32))


def kernel(x, edge_index, bn1_gamma, bn1_beta, W1, b1,
           bn2_gamma, bn2_beta, W2, b2):
    row = edge_index[0].astype(_i32)
    col = edge_index[1].astype(_i32)
    pad = SLOTS - EPW
    col_p = jnp.pad(col.reshape(NW, EPW), ((0, 0), (0, pad))
                    ).reshape(NW * NCH3, CH3)
    row_p = jnp.pad(row.reshape(NW, EPW), ((0, 0), (0, pad)),
                    constant_values=N).reshape(NW * NCH3, CH3)
    g1 = bn1_gamma.reshape(1, D)
    be1 = bn1_beta.reshape(1, D)
    g2r = bn2_gamma.reshape(1, D)
    be2 = bn2_beta.reshape(1, D)
    b1r = b1.reshape(1, D)
    b2r = b2.reshape(1, D)

    m1, r1 = _bn_stats(x)
    degp = _deg_sc(row)
    xb, dinv, u1 = _scale(x, m1, r1, g1, be1, degp, degp)
    s1 = _prop_sc(u1, col_p, row_p)
    g2, acc1 = _t3(s1, s1, dinv, xb, W1[0], W1[1], b1r)
    s2 = _prop_sc(g2, col_p, row_p)
    h = _t4a(s2, s2, dinv, xb, acc1, W1[2])
    m2, r2 = _bn_stats(h)
    hb, g3 = _t4b(h, m2, r2, g2r, be2, dinv)
    s3 = _prop_sc(g3, col_p, row_p)
    acc2, g4 = _t5(s3, s3, dinv, hb, W2[0], W2[1], b2r)
    s4 = _prop_sc(g4, col_p, row_p)
    return _t6(s4, s4, dinv, hb, acc2, xb, W2[2])


# speedup vs baseline: 1.8363x; 1.5896x over previous
import functools

import jax
import jax.numpy as jnp
from jax import lax
from jax.experimental import pallas as pl
from jax.experimental.pallas import tpu as pltpu
from jax.experimental.pallas import tpu_sc as plsc

N = 10000
D = 128
E = 320000
NC = 2
NS = 16
NW = NC * NS
EPW = E // NW
CHUNK = 128
NFULL = EPW // CHUNK
TAIL = EPW - NFULL * CHUNK
RPT = N // NS
ZROWS = 25
WRows = 640
WLAST = N - 15 * WRows

_mesh = plsc.VectorSubcoreMesh(core_axis_name="c", subcore_axis_name="s")

_f32 = jnp.float32
_i32 = jnp.int32


@functools.partial(
    pl.kernel,
    mesh=_mesh,
    out_type=jax.ShapeDtypeStruct((NC * N, D), _f32),
    scratch_types=[
        pltpu.VMEM((CHUNK,), _i32),
        pltpu.VMEM((TAIL,), _i32),
        pltpu.VMEM((CHUNK, D), _f32),
        pltpu.VMEM((TAIL, D), _f32),
        pltpu.VMEM((ZROWS, D), _f32),
        pltpu.VMEM_SHARED((N, D), _f32),
    ],
)
def _deg_sc(row_hbm, out_hbm, rowv, rowt, onesb, onest, zbuf, acc):
    c = lax.axis_index("c")
    s = lax.axis_index("s")
    w = s * NC + c
    ones16 = jnp.ones((16,), _f32)
    zeros16 = jnp.zeros((16,), _f32)

    def fill(i, _):
        for j in range(D // 16):
            onesb[i, pl.ds(j * 16, 16)] = ones16
            zbuf[lax.rem(i, ZROWS), pl.ds(j * 16, 16)] = zeros16
        return 0
    lax.fori_loop(0, CHUNK, fill, 0)

    def fill_t(i, _):
        for j in range(D // 16):
            onest[i, pl.ds(j * 16, 16)] = ones16
        return 0
    lax.fori_loop(0, TAIL, fill_t, 0)

    def zero(i, _):
        pltpu.sync_copy(zbuf, acc.at[pl.ds(s * RPT + i * ZROWS, ZROWS)])
        return 0
    lax.fori_loop(0, RPT // ZROWS, zero, 0)
    plsc.subcore_barrier()

    base = w * EPW

    def step(j, _):
        pltpu.sync_copy(row_hbm.at[pl.ds(base + j * CHUNK, CHUNK)], rowv)
        pltpu.sync_copy(onesb, acc.at[rowv], add=True)
        return 0
    lax.fori_loop(0, NFULL, step, 0)
    pltpu.sync_copy(row_hbm.at[pl.ds(base + NFULL * CHUNK, TAIL)], rowt)
    pltpu.sync_copy(onest, acc.at[rowt], add=True)
    plsc.subcore_barrier()

    @pl.when(s < NS - 1)
    def _():
        pltpu.sync_copy(acc.at[pl.ds(s * WRows, WRows)],
                        out_hbm.at[pl.ds(c * N + s * WRows, WRows)])

    @pl.when(s == NS - 1)
    def _():
        pltpu.sync_copy(acc.at[pl.ds(15 * WRows, WLAST)],
                        out_hbm.at[pl.ds(c * N + 15 * WRows, WLAST)])


@functools.partial(
    pl.kernel,
    mesh=_mesh,
    out_type=jax.ShapeDtypeStruct((NC * N, D), _f32),
    scratch_types=[
        pltpu.VMEM((CHUNK,), _i32),
        pltpu.VMEM((CHUNK,), _i32),
        pltpu.VMEM((TAIL,), _i32),
        pltpu.VMEM((TAIL,), _i32),
        pltpu.VMEM((CHUNK, D), _f32),
        pltpu.VMEM((TAIL, D), _f32),
        pltpu.VMEM((ZROWS, D), _f32),
        pltpu.VMEM_SHARED((N, D), _f32),
        pltpu.SemaphoreType.DMA,
    ],
)
def _prop_sc(u_hbm, col_hbm, row_hbm, out_hbm,
             colv, rowv, colt, rowt, buf, buft, zbuf, acc, sem):
    c = lax.axis_index("c")
    s = lax.axis_index("s")
    w = s * NC + c
    zeros16 = jnp.zeros((16,), _f32)

    def zfill(i, _):
        for j in range(D // 16):
            zbuf[i, pl.ds(j * 16, 16)] = zeros16
        return 0
    lax.fori_loop(0, ZROWS, zfill, 0)

    def zero(i, _):
        pltpu.sync_copy(zbuf, acc.at[pl.ds(s * RPT + i * ZROWS, ZROWS)])
        return 0
    lax.fori_loop(0, RPT // ZROWS, zero, 0)
    plsc.subcore_barrier()

    base = w * EPW

    def step(j, _):
        off = base + j * CHUNK
        pltpu.sync_copy(col_hbm.at[pl.ds(off, CHUNK)], colv)
        pltpu.sync_copy(row_hbm.at[pl.ds(off, CHUNK)], rowv)
        pltpu.async_copy(u_hbm.at[colv], buf, sem).wait()
        pltpu.sync_copy(buf, acc.at[rowv], add=True)
        return 0
    lax.fori_loop(0, NFULL, step, 0)

    offt = base + NFULL * CHUNK
    pltpu.sync_copy(col_hbm.at[pl.ds(offt, TAIL)], colt)
    pltpu.sync_copy(row_hbm.at[pl.ds(offt, TAIL)], rowt)
    pltpu.async_copy(u_hbm.at[colt], buft, sem).wait()
    pltpu.sync_copy(buft, acc.at[rowt], add=True)
    plsc.subcore_barrier()

    @pl.when(s < NS - 1)
    def _():
        pltpu.sync_copy(acc.at[pl.ds(s * WRows, WRows)],
                        out_hbm.at[pl.ds(c * N + s * WRows, WRows)])

    @pl.when(s == NS - 1)
    def _():
        pltpu.sync_copy(acc.at[pl.ds(15 * WRows, WLAST)],
                        out_hbm.at[pl.ds(c * N + 15 * WRows, WLAST)])


BM = 2000
GRID = N // BM

_sd = jax.ShapeDtypeStruct


def _dot(a, b):
    return jax.lax.dot_general(a, b, (((1,), (0,)), ((), ())),
                               precision=jax.lax.Precision.HIGHEST,
                               preferred_element_type=_f32)


def _stats_body(x_ref, m_ref, r_ref):
    x = x_ref[...]
    m = jnp.mean(x, axis=0, keepdims=True)
    m_ref[...] = m
    v = jnp.mean((x - m) * (x - m), axis=0, keepdims=True)
    r_ref[...] = jax.lax.rsqrt(v + 1e-5)


_bn_stats = pl.pallas_call(
    _stats_body,
    out_shape=(_sd((1, D), _f32), _sd((1, D), _f32)))


def _blk(shape):
    return pl.BlockSpec(shape, lambda i: (i, 0))


def _blk2(shape):
    return pl.BlockSpec(shape, lambda i: (i + GRID, 0))


def _full(shape):
    return pl.BlockSpec(shape, lambda i: (0, 0))


def _ones_body(m_ref, o_ref):
    o_ref[...] = jnp.broadcast_to(m_ref[...] * 0.0 + 1.0, (BM, D))


_ones_tc = pl.pallas_call(
    _ones_body,
    grid=(GRID,),
    in_specs=[_full((1, D))],
    out_specs=_blk((BM, D)),
    out_shape=_sd((N, D), _f32))


def _scale_body(x_ref, m_ref, r_ref, g_ref, be_ref, dga_ref, dgb_ref,
                xb_ref, dinv_ref, u1_ref):
    xb = ((x_ref[...] - m_ref[...]) * r_ref[...] * g_ref[...]
          + be_ref[...])
    xb_ref[...] = xb
    deg = dga_ref[:, 0:1] + dgb_ref[:, 0:1]
    dinv = jnp.where(deg > 0.0,
                     jax.lax.rsqrt(jnp.maximum(deg, 1.0)), 0.0)
    dinv_ref[...] = dinv
    u1_ref[...] = xb * dinv


_scale = pl.pallas_call(
    _scale_body,
    grid=(GRID,),
    in_specs=[_blk((BM, D)), _full((1, D)), _full((1, D)),
              _full((1, D)), _full((1, D)),
              _blk((BM, D)), _blk2((BM, D))],
    out_specs=(_blk((BM, D)), _blk((BM, 1)), _blk((BM, D))),
    out_shape=(_sd((N, D), _f32), _sd((N, 1), _f32), _sd((N, D), _f32)))


def _t3_body(sa_ref, sb_ref, dinv_ref, xb_ref, w0_ref, w1_ref, b_ref,
             g2_ref, acc1_ref):
    dinv = dinv_ref[...]
    tx1 = -dinv * (sa_ref[...] + sb_ref[...])
    g2_ref[...] = dinv * tx1
    acc1_ref[...] = (_dot(xb_ref[...], w0_ref[...])
                     + _dot(tx1, w1_ref[...]) + b_ref[...])


_t3 = pl.pallas_call(
    _t3_body,
    grid=(GRID,),
    in_specs=[_blk((BM, D)), _blk2((BM, D)), _blk((BM, 1)), _blk((BM, D)),
              _full((D, D)), _full((D, D)), _full((1, D))],
    out_specs=(_blk((BM, D)), _blk((BM, D))),
    out_shape=(_sd((N, D), _f32), _sd((N, D), _f32)))


def _t4a_body(sa_ref, sb_ref, dinv_ref, xb_ref, acc1_ref, w2_ref,
              h_ref):
    tx2 = (-2.0 * dinv_ref[...] * (sa_ref[...] + sb_ref[...])
           - xb_ref[...])
    h_ref[...] = jnp.maximum(acc1_ref[...] + _dot(tx2, w2_ref[...]), 0.0)


_t4a = pl.pallas_call(
    _t4a_body,
    grid=(GRID,),
    in_specs=[_blk((BM, D)), _blk2((BM, D)), _blk((BM, 1)), _blk((BM, D)),
              _blk((BM, D)), _full((D, D))],
    out_specs=_blk((BM, D)),
    out_shape=_sd((N, D), _f32))


def _t4b_body(h_ref, m_ref, r_ref, g_ref, be_ref, dinv_ref,
              hb_ref, g3_ref):
    hb = ((h_ref[...] - m_ref[...]) * r_ref[...] * g_ref[...]
          + be_ref[...])
    hb_ref[...] = hb
    g3_ref[...] = dinv_ref[...] * hb


_t4b = pl.pallas_call(
    _t4b_body,
    grid=(GRID,),
    in_specs=[_blk((BM, D)), _full((1, D)), _full((1, D)),
              _full((1, D)), _full((1, D)), _blk((BM, 1))],
    out_specs=(_blk((BM, D)), _blk((BM, D))),
    out_shape=(_sd((N, D), _f32), _sd((N, D), _f32)))


def _t5_body(sa_ref, sb_ref, dinv_ref, hb_ref, w0_ref, w1_ref, b_ref,
             acc2_ref, g4_ref):
    dinv = dinv_ref[...]
    ty1 = -dinv * (sa_ref[...] + sb_ref[...])
    g4_ref[...] = dinv * ty1
    acc2_ref[...] = (_dot(hb_ref[...], w0_ref[...])
                     + _dot(ty1, w1_ref[...]) + b_ref[...])


_t5 = pl.pallas_call(
    _t5_body,
    grid=(GRID,),
    in_specs=[_blk((BM, D)), _blk2((BM, D)), _blk((BM, 1)), _blk((BM, D)),
              _full((D, D)), _full((D, D)), _full((1, D))],
    out_specs=(_blk((BM, D)), _blk((BM, D))),
    out_shape=(_sd((N, D), _f32), _sd((N, D), _f32)))


def _t6_body(sa_ref, sb_ref, dinv_ref, hb_ref, acc2_ref, xb_ref, w2_ref,
             o_ref):
    ty2 = (-2.0 * dinv_ref[...] * (sa_ref[...] + sb_ref[...])
           - hb_ref[...])
    o_ref[...] = jnp.maximum(
        xb_ref[...] + acc2_ref[...] + _dot(ty2, w2_ref[...]), 0.0)


_t6 = pl.pallas_call(
    _t6_body,
    grid=(GRID,),
    in_specs=[_blk((BM, D)), _blk2((BM, D)), _blk((BM, 1)), _blk((BM, D)),
              _blk((BM, D)), _blk((BM, D)), _full((D, D))],
    out_specs=_blk((BM, D)),
    out_shape=_sd((N, D), _f32))


def kernel(x, edge_index, bn1_gamma, bn1_beta, W1, b1,
           bn2_gamma, bn2_beta, W2, b2):
    row = edge_index[0].astype(_i32)
    col = edge_index[1].astype(_i32)
    g1 = bn1_gamma.reshape(1, D)
    be1 = bn1_beta.reshape(1, D)
    g2r = bn2_gamma.reshape(1, D)
    be2 = bn2_beta.reshape(1, D)
    b1r = b1.reshape(1, D)
    b2r = b2.reshape(1, D)

    m1, r1 = _bn_stats(x)
    degp = _deg_sc(row)
    xb, dinv, u1 = _scale(x, m1, r1, g1, be1, degp, degp)
    s1 = _prop_sc(u1, col, row)
    g2, acc1 = _t3(s1, s1, dinv, xb, W1[0], W1[1], b1r)
    s2 = _prop_sc(g2, col, row)
    h = _t4a(s2, s2, dinv, xb, acc1, W1[2])
    m2, r2 = _bn_stats(h)
    hb, g3 = _t4b(h, m2, r2, g2r, be2, dinv)
    s3 = _prop_sc(g3, col, row)
    acc2, g4 = _t5(s3, s3, dinv, hb, W2[0], W2[1], b2r)
    s4 = _prop_sc(g4, col, row)
    return _t6(s4, s4, dinv, hb, acc2, xb, W2[2])


# speedup vs baseline: 2.1872x; 1.1911x over previous
import functools

import jax
import jax.numpy as jnp
from jax import lax
from jax.experimental import pallas as pl
from jax.experimental.pallas import tpu as pltpu
from jax.experimental.pallas import tpu_sc as plsc

N = 10000
D = 128
E = 320000
NC = 2
NS = 16
NW = NC * NS
EPW = E // NW
CHUNK = 128
NFULL = EPW // CHUNK
TAIL = EPW - NFULL * CHUNK
RPT = N // NS
ZROWS = 25
WRows = 640
WLAST = N - 15 * WRows

_mesh = plsc.VectorSubcoreMesh(core_axis_name="c", subcore_axis_name="s")

_f32 = jnp.float32
_i32 = jnp.int32


@functools.partial(
    pl.kernel,
    mesh=_mesh,
    out_type=jax.ShapeDtypeStruct((NC * N, D), _f32),
    scratch_types=[
        pltpu.VMEM((CHUNK,), _i32),
        pltpu.VMEM((TAIL,), _i32),
        pltpu.VMEM((CHUNK, D), _f32),
        pltpu.VMEM((TAIL, D), _f32),
        pltpu.VMEM((ZROWS, D), _f32),
        pltpu.VMEM_SHARED((N, D), _f32),
    ],
)
def _deg_sc(row_hbm, out_hbm, rowv, rowt, onesb, onest, zbuf, acc):
    c = lax.axis_index("c")
    s = lax.axis_index("s")
    w = s * NC + c
    ones16 = jnp.ones((16,), _f32)
    zeros16 = jnp.zeros((16,), _f32)

    def fill(i, _):
        for j in range(D // 16):
            onesb[i, pl.ds(j * 16, 16)] = ones16
            zbuf[lax.rem(i, ZROWS), pl.ds(j * 16, 16)] = zeros16
        return 0
    lax.fori_loop(0, CHUNK, fill, 0)

    def fill_t(i, _):
        for j in range(D // 16):
            onest[i, pl.ds(j * 16, 16)] = ones16
        return 0
    lax.fori_loop(0, TAIL, fill_t, 0)

    def zero(i, _):
        pltpu.sync_copy(zbuf, acc.at[pl.ds(s * RPT + i * ZROWS, ZROWS)])
        return 0
    lax.fori_loop(0, RPT // ZROWS, zero, 0)
    plsc.subcore_barrier()

    base = w * EPW

    def step(j, _):
        pltpu.sync_copy(row_hbm.at[pl.ds(base + j * CHUNK, CHUNK)], rowv)
        pltpu.sync_copy(onesb, acc.at[rowv], add=True)
        return 0
    lax.fori_loop(0, NFULL, step, 0)
    pltpu.sync_copy(row_hbm.at[pl.ds(base + NFULL * CHUNK, TAIL)], rowt)
    pltpu.sync_copy(onest, acc.at[rowt], add=True)
    plsc.subcore_barrier()

    @pl.when(s < NS - 1)
    def _():
        pltpu.sync_copy(acc.at[pl.ds(s * WRows, WRows)],
                        out_hbm.at[pl.ds(c * N + s * WRows, WRows)])

    @pl.when(s == NS - 1)
    def _():
        pltpu.sync_copy(acc.at[pl.ds(15 * WRows, WLAST)],
                        out_hbm.at[pl.ds(c * N + 15 * WRows, WLAST)])


@functools.partial(
    pl.kernel,
    mesh=_mesh,
    out_type=jax.ShapeDtypeStruct((NC * N, D), _f32),
    scratch_types=[
        pltpu.VMEM((CHUNK,), _i32),
        pltpu.VMEM((CHUNK,), _i32),
        pltpu.VMEM((CHUNK,), _i32),
        pltpu.VMEM((TAIL,), _i32),
        pltpu.VMEM((TAIL,), _i32),
        pltpu.VMEM((CHUNK, D), _f32),
        pltpu.VMEM((CHUNK, D), _f32),
        pltpu.VMEM((TAIL, D), _f32),
        pltpu.VMEM((ZROWS, D), _f32),
        pltpu.VMEM_SHARED((N, D), _f32),
        pltpu.SemaphoreType.DMA,
        pltpu.SemaphoreType.DMA,
        pltpu.SemaphoreType.DMA,
    ],
)
def _prop_sc(u_hbm, col_hbm, row_hbm, out_hbm,
             colv, rowv0, rowv1, colt, rowt, b0, b1, buft, zbuf, acc,
             sem, ssem0, ssem1):
    c = lax.axis_index("c")
    s = lax.axis_index("s")
    w = s * NC + c
    zeros16 = jnp.zeros((16,), _f32)

    def zfill(i, _):
        for j in range(D // 16):
            zbuf[i, pl.ds(j * 16, 16)] = zeros16
        return 0
    lax.fori_loop(0, ZROWS, zfill, 0)

    def zero(i, _):
        pltpu.sync_copy(zbuf, acc.at[pl.ds(s * RPT + i * ZROWS, ZROWS)])
        return 0
    lax.fori_loop(0, RPT // ZROWS, zero, 0)
    plsc.subcore_barrier()

    base = w * EPW

    def _sdrain(ssem, buf):
        pltpu.make_async_copy(u_hbm.at[pl.ds(0, CHUNK)], buf, ssem).wait()

    def step(g, _):
        off = base + 2 * g * CHUNK
        pltpu.sync_copy(col_hbm.at[pl.ds(off, CHUNK)], colv)
        pltpu.sync_copy(row_hbm.at[pl.ds(off, CHUNK)], rowv0)
        pltpu.async_copy(u_hbm.at[colv], b0, sem).wait()

        @pl.when(g > 0)
        def _():
            _sdrain(ssem1, b1)
        pltpu.async_copy(b0, acc.at[rowv0], ssem0, add=True)
        pltpu.sync_copy(col_hbm.at[pl.ds(off + CHUNK, CHUNK)], colv)
        pltpu.sync_copy(row_hbm.at[pl.ds(off + CHUNK, CHUNK)], rowv1)
        pltpu.async_copy(u_hbm.at[colv], b1, sem).wait()
        _sdrain(ssem0, b0)
        pltpu.async_copy(b1, acc.at[rowv1], ssem1, add=True)
        return 0
    lax.fori_loop(0, NFULL // 2, step, 0)
    _sdrain(ssem1, b1)

    offt = base + NFULL * CHUNK
    pltpu.sync_copy(col_hbm.at[pl.ds(offt, TAIL)], colt)
    pltpu.sync_copy(row_hbm.at[pl.ds(offt, TAIL)], rowt)
    pltpu.async_copy(u_hbm.at[colt], buft, sem).wait()
    pltpu.sync_copy(buft, acc.at[rowt], add=True)
    plsc.subcore_barrier()

    @pl.when(s < NS - 1)
    def _():
        pltpu.sync_copy(acc.at[pl.ds(s * WRows, WRows)],
                        out_hbm.at[pl.ds(c * N + s * WRows, WRows)])

    @pl.when(s == NS - 1)
    def _():
        pltpu.sync_copy(acc.at[pl.ds(15 * WRows, WLAST)],
                        out_hbm.at[pl.ds(c * N + 15 * WRows, WLAST)])


BM = 2000
GRID = N // BM

_sd = jax.ShapeDtypeStruct


def _dot(a, b):
    return jax.lax.dot_general(a, b, (((1,), (0,)), ((), ())),
                               precision=jax.lax.Precision.HIGHEST,
                               preferred_element_type=_f32)


def _stats_body(x_ref, m_ref, r_ref):
    x = x_ref[...]
    m = jnp.mean(x, axis=0, keepdims=True)
    m_ref[...] = m
    v = jnp.mean((x - m) * (x - m), axis=0, keepdims=True)
    r_ref[...] = jax.lax.rsqrt(v + 1e-5)


_bn_stats = pl.pallas_call(
    _stats_body,
    out_shape=(_sd((1, D), _f32), _sd((1, D), _f32)))


def _blk(shape):
    return pl.BlockSpec(shape, lambda i: (i, 0))


def _blk2(shape):
    return pl.BlockSpec(shape, lambda i: (i + GRID, 0))


def _full(shape):
    return pl.BlockSpec(shape, lambda i: (0, 0))


def _ones_body(m_ref, o_ref):
    o_ref[...] = jnp.broadcast_to(m_ref[...] * 0.0 + 1.0, (BM, D))


_ones_tc = pl.pallas_call(
    _ones_body,
    grid=(GRID,),
    in_specs=[_full((1, D))],
    out_specs=_blk((BM, D)),
    out_shape=_sd((N, D), _f32))


def _scale_body(x_ref, m_ref, r_ref, g_ref, be_ref, dga_ref, dgb_ref,
                xb_ref, dinv_ref, u1_ref):
    xb = ((x_ref[...] - m_ref[...]) * r_ref[...] * g_ref[...]
          + be_ref[...])
    xb_ref[...] = xb
    deg = dga_ref[:, 0:1] + dgb_ref[:, 0:1]
    dinv = jnp.where(deg > 0.0,
                     jax.lax.rsqrt(jnp.maximum(deg, 1.0)), 0.0)
    dinv_ref[...] = dinv
    u1_ref[...] = xb * dinv


_scale = pl.pallas_call(
    _scale_body,
    grid=(GRID,),
    in_specs=[_blk((BM, D)), _full((1, D)), _full((1, D)),
              _full((1, D)), _full((1, D)),
              _blk((BM, D)), _blk2((BM, D))],
    out_specs=(_blk((BM, D)), _blk((BM, 1)), _blk((BM, D))),
    out_shape=(_sd((N, D), _f32), _sd((N, 1), _f32), _sd((N, D), _f32)))


def _t3_body(sa_ref, sb_ref, dinv_ref, xb_ref, w0_ref, w1_ref, b_ref,
             g2_ref, acc1_ref):
    dinv = dinv_ref[...]
    tx1 = -dinv * (sa_ref[...] + sb_ref[...])
    g2_ref[...] = dinv * tx1
    acc1_ref[...] = (_dot(xb_ref[...], w0_ref[...])
                     + _dot(tx1, w1_ref[...]) + b_ref[...])


_t3 = pl.pallas_call(
    _t3_body,
    grid=(GRID,),
    in_specs=[_blk((BM, D)), _blk2((BM, D)), _blk((BM, 1)), _blk((BM, D)),
              _full((D, D)), _full((D, D)), _full((1, D))],
    out_specs=(_blk((BM, D)), _blk((BM, D))),
    out_shape=(_sd((N, D), _f32), _sd((N, D), _f32)))


def _t4a_body(sa_ref, sb_ref, dinv_ref, xb_ref, acc1_ref, w2_ref,
              h_ref):
    tx2 = (-2.0 * dinv_ref[...] * (sa_ref[...] + sb_ref[...])
           - xb_ref[...])
    h_ref[...] = jnp.maximum(acc1_ref[...] + _dot(tx2, w2_ref[...]), 0.0)


_t4a = pl.pallas_call(
    _t4a_body,
    grid=(GRID,),
    in_specs=[_blk((BM, D)), _blk2((BM, D)), _blk((BM, 1)), _blk((BM, D)),
              _blk((BM, D)), _full((D, D))],
    out_specs=_blk((BM, D)),
    out_shape=_sd((N, D), _f32))


def _t4b_body(h_ref, m_ref, r_ref, g_ref, be_ref, dinv_ref,
              hb_ref, g3_ref):
    hb = ((h_ref[...] - m_ref[...]) * r_ref[...] * g_ref[...]
          + be_ref[...])
    hb_ref[...] = hb
    g3_ref[...] = dinv_ref[...] * hb


_t4b = pl.pallas_call(
    _t4b_body,
    grid=(GRID,),
    in_specs=[_blk((BM, D)), _full((1, D)), _full((1, D)),
              _full((1, D)), _full((1, D)), _blk((BM, 1))],
    out_specs=(_blk((BM, D)), _blk((BM, D))),
    out_shape=(_sd((N, D), _f32), _sd((N, D), _f32)))


def _t5_body(sa_ref, sb_ref, dinv_ref, hb_ref, w0_ref, w1_ref, b_ref,
             acc2_ref, g4_ref):
    dinv = dinv_ref[...]
    ty1 = -dinv * (sa_ref[...] + sb_ref[...])
    g4_ref[...] = dinv * ty1
    acc2_ref[...] = (_dot(hb_ref[...], w0_ref[...])
                     + _dot(ty1, w1_ref[...]) + b_ref[...])


_t5 = pl.pallas_call(
    _t5_body,
    grid=(GRID,),
    in_specs=[_blk((BM, D)), _blk2((BM, D)), _blk((BM, 1)), _blk((BM, D)),
              _full((D, D)), _full((D, D)), _full((1, D))],
    out_specs=(_blk((BM, D)), _blk((BM, D))),
    out_shape=(_sd((N, D), _f32), _sd((N, D), _f32)))


def _t6_body(sa_ref, sb_ref, dinv_ref, hb_ref, acc2_ref, xb_ref, w2_ref,
             o_ref):
    ty2 = (-2.0 * dinv_ref[...] * (sa_ref[...] + sb_ref[...])
           - hb_ref[...])
    o_ref[...] = jnp.maximum(
        xb_ref[...] + acc2_ref[...] + _dot(ty2, w2_ref[...]), 0.0)


_t6 = pl.pallas_call(
    _t6_body,
    grid=(GRID,),
    in_specs=[_blk((BM, D)), _blk2((BM, D)), _blk((BM, 1)), _blk((BM, D)),
              _blk((BM, D)), _blk((BM, D)), _full((D, D))],
    out_specs=_blk((BM, D)),
    out_shape=_sd((N, D), _f32))


def kernel(x, edge_index, bn1_gamma, bn1_beta, W1, b1,
           bn2_gamma, bn2_beta, W2, b2):
    row = edge_index[0].astype(_i32)
    col = edge_index[1].astype(_i32)
    g1 = bn1_gamma.reshape(1, D)
    be1 = bn1_beta.reshape(1, D)
    g2r = bn2_gamma.reshape(1, D)
    be2 = bn2_beta.reshape(1, D)
    b1r = b1.reshape(1, D)
    b2r = b2.reshape(1, D)

    m1, r1 = _bn_stats(x)
    degp = _deg_sc(row)
    xb, dinv, u1 = _scale(x, m1, r1, g1, be1, degp, degp)
    s1 = _prop_sc(u1, col, row)
    g2, acc1 = _t3(s1, s1, dinv, xb, W1[0], W1[1], b1r)
    s2 = _prop_sc(g2, col, row)
    h = _t4a(s2, s2, dinv, xb, acc1, W1[2])
    m2, r2 = _bn_stats(h)
    hb, g3 = _t4b(h, m2, r2, g2r, be2, dinv)
    s3 = _prop_sc(g3, col, row)
    acc2, g4 = _t5(s3, s3, dinv, hb, W2[0], W2[1], b2r)
    s4 = _prop_sc(g4, col, row)
    return _t6(s4, s4, dinv, hb, acc2, xb, W2[2])


# speedup vs baseline: 2.5324x; 1.1578x over previous
import functools

import jax
import jax.numpy as jnp
from jax import lax
from jax.experimental import pallas as pl
from jax.experimental.pallas import tpu as pltpu
from jax.experimental.pallas import tpu_sc as plsc

N = 10000
D = 128
E = 320000
NC = 2
NS = 16
NW = NC * NS
EPW = E // NW
CHUNK = 128
NFULL = EPW // CHUNK
TAIL = EPW - NFULL * CHUNK
RPT = N // NS
ZROWS = 25
WRows = 640
WLAST = N - 15 * WRows

_mesh = plsc.VectorSubcoreMesh(core_axis_name="c", subcore_axis_name="s")

_f32 = jnp.float32
_i32 = jnp.int32


@functools.partial(
    pl.kernel,
    mesh=_mesh,
    out_type=jax.ShapeDtypeStruct((NC * N, D), _f32),
    scratch_types=[
        pltpu.VMEM((CHUNK,), _i32),
        pltpu.VMEM((TAIL,), _i32),
        pltpu.VMEM((CHUNK, D), _f32),
        pltpu.VMEM((TAIL, D), _f32),
        pltpu.VMEM((ZROWS, D), _f32),
        pltpu.VMEM_SHARED((N, D), _f32),
    ],
)
def _deg_sc(row_hbm, out_hbm, rowv, rowt, onesb, onest, zbuf, acc):
    c = lax.axis_index("c")
    s = lax.axis_index("s")
    w = s * NC + c
    ones16 = jnp.ones((16,), _f32)
    zeros16 = jnp.zeros((16,), _f32)

    def fill(i, _):
        for j in range(D // 16):
            onesb[i, pl.ds(j * 16, 16)] = ones16
            zbuf[lax.rem(i, ZROWS), pl.ds(j * 16, 16)] = zeros16
        return 0
    lax.fori_loop(0, CHUNK, fill, 0)

    def fill_t(i, _):
        for j in range(D // 16):
            onest[i, pl.ds(j * 16, 16)] = ones16
        return 0
    lax.fori_loop(0, TAIL, fill_t, 0)

    def zero(i, _):
        pltpu.sync_copy(zbuf, acc.at[pl.ds(s * RPT + i * ZROWS, ZROWS)])
        return 0
    lax.fori_loop(0, RPT // ZROWS, zero, 0)
    plsc.subcore_barrier()

    base = w * EPW

    def step(j, _):
        pltpu.sync_copy(row_hbm.at[pl.ds(base + j * CHUNK, CHUNK)], rowv)
        pltpu.sync_copy(onesb, acc.at[rowv], add=True)
        return 0
    lax.fori_loop(0, NFULL, step, 0)
    pltpu.sync_copy(row_hbm.at[pl.ds(base + NFULL * CHUNK, TAIL)], rowt)
    pltpu.sync_copy(onest, acc.at[rowt], add=True)
    plsc.subcore_barrier()

    @pl.when(s < NS - 1)
    def _():
        pltpu.sync_copy(acc.at[pl.ds(s * WRows, WRows)],
                        out_hbm.at[pl.ds(c * N + s * WRows, WRows)])

    @pl.when(s == NS - 1)
    def _():
        pltpu.sync_copy(acc.at[pl.ds(15 * WRows, WLAST)],
                        out_hbm.at[pl.ds(c * N + 15 * WRows, WLAST)])


@functools.partial(
    pl.kernel,
    mesh=_mesh,
    out_type=jax.ShapeDtypeStruct((NC * N, D), _f32),
    scratch_types=[
        pltpu.VMEM((CHUNK,), _i32),
        pltpu.VMEM((CHUNK,), _i32),
        pltpu.VMEM((CHUNK,), _i32),
        pltpu.VMEM((CHUNK,), _i32),
        pltpu.VMEM((TAIL,), _i32),
        pltpu.VMEM((TAIL,), _i32),
        pltpu.VMEM((CHUNK, D), _f32),
        pltpu.VMEM((CHUNK, D), _f32),
        pltpu.VMEM((TAIL, D), _f32),
        pltpu.VMEM((ZROWS, D), _f32),
        pltpu.VMEM_SHARED((N, D), _f32),
        pltpu.SemaphoreType.DMA,
        pltpu.SemaphoreType.DMA,
        pltpu.SemaphoreType.DMA,
    ],
)
def _prop_sc(u_hbm, col_hbm, row_hbm, out_hbm,
             colv0, colv1, rowv0, rowv1, colt, rowt, b0, b1, buft, zbuf,
             acc, sem, ssem0, ssem1):
    c = lax.axis_index("c")
    s = lax.axis_index("s")
    w = s * NC + c
    zeros16 = jnp.zeros((16,), _f32)

    def zfill(i, _):
        for j in range(D // 16):
            zbuf[i, pl.ds(j * 16, 16)] = zeros16
        return 0
    lax.fori_loop(0, ZROWS, zfill, 0)

    def zero(i, _):
        pltpu.sync_copy(zbuf, acc.at[pl.ds(s * RPT + i * ZROWS, ZROWS)])
        return 0
    lax.fori_loop(0, RPT // ZROWS, zero, 0)
    plsc.subcore_barrier()

    base = w * EPW

    def _sdrain(ssem, buf):
        pltpu.make_async_copy(u_hbm.at[pl.ds(0, CHUNK)], buf, ssem).wait()

    def _gwait(buf):
        pltpu.make_async_copy(u_hbm.at[pl.ds(0, CHUNK)], buf, sem).wait()

    pltpu.sync_copy(col_hbm.at[pl.ds(base, CHUNK)], colv0)
    pltpu.sync_copy(row_hbm.at[pl.ds(base, CHUNK)], rowv0)
    pltpu.async_copy(u_hbm.at[colv0], b0, sem)

    def step(g, _):
        off = base + 2 * g * CHUNK
        pltpu.sync_copy(col_hbm.at[pl.ds(off + CHUNK, CHUNK)], colv1)
        pltpu.sync_copy(row_hbm.at[pl.ds(off + CHUNK, CHUNK)], rowv1)
        _gwait(b0)

        @pl.when(g > 0)
        def _():
            _sdrain(ssem1, b1)
        pltpu.async_copy(u_hbm.at[colv1], b1, sem)
        pltpu.async_copy(b0, acc.at[rowv0], ssem0, add=True)
        _gwait(b1)
        _sdrain(ssem0, b0)

        @pl.when(g < NFULL // 2 - 1)
        def _():
            offn = off + 2 * CHUNK
            pltpu.sync_copy(col_hbm.at[pl.ds(offn, CHUNK)], colv0)
            pltpu.sync_copy(row_hbm.at[pl.ds(offn, CHUNK)], rowv0)
            pltpu.async_copy(u_hbm.at[colv0], b0, sem)
        pltpu.async_copy(b1, acc.at[rowv1], ssem1, add=True)
        return 0
    lax.fori_loop(0, NFULL // 2, step, 0)
    _sdrain(ssem1, b1)

    offt = base + NFULL * CHUNK
    pltpu.sync_copy(col_hbm.at[pl.ds(offt, TAIL)], colt)
    pltpu.sync_copy(row_hbm.at[pl.ds(offt, TAIL)], rowt)
    pltpu.async_copy(u_hbm.at[colt], buft, sem).wait()
    pltpu.sync_copy(buft, acc.at[rowt], add=True)
    plsc.subcore_barrier()

    @pl.when(s < NS - 1)
    def _():
        pltpu.sync_copy(acc.at[pl.ds(s * WRows, WRows)],
                        out_hbm.at[pl.ds(c * N + s * WRows, WRows)])

    @pl.when(s == NS - 1)
    def _():
        pltpu.sync_copy(acc.at[pl.ds(15 * WRows, WLAST)],
                        out_hbm.at[pl.ds(c * N + 15 * WRows, WLAST)])


BM = 2000
GRID = N // BM

_sd = jax.ShapeDtypeStruct


def _dot(a, b):
    return jax.lax.dot_general(a, b, (((1,), (0,)), ((), ())),
                               precision=jax.lax.Precision.HIGHEST,
                               preferred_element_type=_f32)


def _stats_body(x_ref, m_ref, r_ref):
    x = x_ref[...]
    m = jnp.mean(x, axis=0, keepdims=True)
    m_ref[...] = m
    v = jnp.mean((x - m) * (x - m), axis=0, keepdims=True)
    r_ref[...] = jax.lax.rsqrt(v + 1e-5)


_bn_stats = pl.pallas_call(
    _stats_body,
    out_shape=(_sd((1, D), _f32), _sd((1, D), _f32)))


def _blk(shape):
    return pl.BlockSpec(shape, lambda i: (i, 0))


def _blk2(shape):
    return pl.BlockSpec(shape, lambda i: (i + GRID, 0))


def _full(shape):
    return pl.BlockSpec(shape, lambda i: (0, 0))


def _ones_body(m_ref, o_ref):
    o_ref[...] = jnp.broadcast_to(m_ref[...] * 0.0 + 1.0, (BM, D))


_ones_tc = pl.pallas_call(
    _ones_body,
    grid=(GRID,),
    in_specs=[_full((1, D))],
    out_specs=_blk((BM, D)),
    out_shape=_sd((N, D), _f32))


def _scale_body(x_ref, m_ref, r_ref, g_ref, be_ref, dga_ref, dgb_ref,
                xb_ref, dinv_ref, u1_ref):
    xb = ((x_ref[...] - m_ref[...]) * r_ref[...] * g_ref[...]
          + be_ref[...])
    xb_ref[...] = xb
    deg = dga_ref[:, 0:1] + dgb_ref[:, 0:1]
    dinv = jnp.where(deg > 0.0,
                     jax.lax.rsqrt(jnp.maximum(deg, 1.0)), 0.0)
    dinv_ref[...] = dinv
    u1_ref[...] = xb * dinv


_scale = pl.pallas_call(
    _scale_body,
    grid=(GRID,),
    in_specs=[_blk((BM, D)), _full((1, D)), _full((1, D)),
              _full((1, D)), _full((1, D)),
              _blk((BM, D)), _blk2((BM, D))],
    out_specs=(_blk((BM, D)), _blk((BM, 1)), _blk((BM, D))),
    out_shape=(_sd((N, D), _f32), _sd((N, 1), _f32), _sd((N, D), _f32)))


def _t3_body(sa_ref, sb_ref, dinv_ref, xb_ref, w0_ref, w1_ref, b_ref,
             g2_ref, acc1_ref):
    dinv = dinv_ref[...]
    tx1 = -dinv * (sa_ref[...] + sb_ref[...])
    g2_ref[...] = dinv * tx1
    acc1_ref[...] = (_dot(xb_ref[...], w0_ref[...])
                     + _dot(tx1, w1_ref[...]) + b_ref[...])


_t3 = pl.pallas_call(
    _t3_body,
    grid=(GRID,),
    in_specs=[_blk((BM, D)), _blk2((BM, D)), _blk((BM, 1)), _blk((BM, D)),
              _full((D, D)), _full((D, D)), _full((1, D))],
    out_specs=(_blk((BM, D)), _blk((BM, D))),
    out_shape=(_sd((N, D), _f32), _sd((N, D), _f32)))


def _t4a_body(sa_ref, sb_ref, dinv_ref, xb_ref, acc1_ref, w2_ref,
              h_ref):
    tx2 = (-2.0 * dinv_ref[...] * (sa_ref[...] + sb_ref[...])
           - xb_ref[...])
    h_ref[...] = jnp.maximum(acc1_ref[...] + _dot(tx2, w2_ref[...]), 0.0)


_t4a = pl.pallas_call(
    _t4a_body,
    grid=(GRID,),
    in_specs=[_blk((BM, D)), _blk2((BM, D)), _blk((BM, 1)), _blk((BM, D)),
              _blk((BM, D)), _full((D, D))],
    out_specs=_blk((BM, D)),
    out_shape=_sd((N, D), _f32))


def _t4b_body(h_ref, m_ref, r_ref, g_ref, be_ref, dinv_ref,
              hb_ref, g3_ref):
    hb = ((h_ref[...] - m_ref[...]) * r_ref[...] * g_ref[...]
          + be_ref[...])
    hb_ref[...] = hb
    g3_ref[...] = dinv_ref[...] * hb


_t4b = pl.pallas_call(
    _t4b_body,
    grid=(GRID,),
    in_specs=[_blk((BM, D)), _full((1, D)), _full((1, D)),
              _full((1, D)), _full((1, D)), _blk((BM, 1))],
    out_specs=(_blk((BM, D)), _blk((BM, D))),
    out_shape=(_sd((N, D), _f32), _sd((N, D), _f32)))


def _t5_body(sa_ref, sb_ref, dinv_ref, hb_ref, w0_ref, w1_ref, b_ref,
             acc2_ref, g4_ref):
    dinv = dinv_ref[...]
    ty1 = -dinv * (sa_ref[...] + sb_ref[...])
    g4_ref[...] = dinv * ty1
    acc2_ref[...] = (_dot(hb_ref[...], w0_ref[...])
                     + _dot(ty1, w1_ref[...]) + b_ref[...])


_t5 = pl.pallas_call(
    _t5_body,
    grid=(GRID,),
    in_specs=[_blk((BM, D)), _blk2((BM, D)), _blk((BM, 1)), _blk((BM, D)),
              _full((D, D)), _full((D, D)), _full((1, D))],
    out_specs=(_blk((BM, D)), _blk((BM, D))),
    out_shape=(_sd((N, D), _f32), _sd((N, D), _f32)))


def _t6_body(sa_ref, sb_ref, dinv_ref, hb_ref, acc2_ref, xb_ref, w2_ref,
             o_ref):
    ty2 = (-2.0 * dinv_ref[...] * (sa_ref[...] + sb_ref[...])
           - hb_ref[...])
    o_ref[...] = jnp.maximum(
        xb_ref[...] + acc2_ref[...] + _dot(ty2, w2_ref[...]), 0.0)


_t6 = pl.pallas_call(
    _t6_body,
    grid=(GRID,),
    in_specs=[_blk((BM, D)), _blk2((BM, D)), _blk((BM, 1)), _blk((BM, D)),
              _blk((BM, D)), _blk((BM, D)), _full((D, D))],
    out_specs=_blk((BM, D)),
    out_shape=_sd((N, D), _f32))


def kernel(x, edge_index, bn1_gamma, bn1_beta, W1, b1,
           bn2_gamma, bn2_beta, W2, b2):
    row = edge_index[0].astype(_i32)
    col = edge_index[1].astype(_i32)
    g1 = bn1_gamma.reshape(1, D)
    be1 = bn1_beta.reshape(1, D)
    g2r = bn2_gamma.reshape(1, D)
    be2 = bn2_beta.reshape(1, D)
    b1r = b1.reshape(1, D)
    b2r = b2.reshape(1, D)

    m1, r1 = _bn_stats(x)
    degp = _deg_sc(row)
    xb, dinv, u1 = _scale(x, m1, r1, g1, be1, degp, degp)
    s1 = _prop_sc(u1, col, row)
    g2, acc1 = _t3(s1, s1, dinv, xb, W1[0], W1[1], b1r)
    s2 = _prop_sc(g2, col, row)
    h = _t4a(s2, s2, dinv, xb, acc1, W1[2])
    m2, r2 = _bn_stats(h)
    hb, g3 = _t4b(h, m2, r2, g2r, be2, dinv)
    s3 = _prop_sc(g3, col, row)
    acc2, g4 = _t5(s3, s3, dinv, hb, W2[0], W2[1], b2r)
    s4 = _prop_sc(g4, col, row)
    return _t6(s4, s4, dinv, hb, acc2, xb, W2[2])


# speedup vs baseline: 2.5405x; 1.0032x over previous
import functools

import jax
import jax.numpy as jnp
from jax import lax
from jax.experimental import pallas as pl
from jax.experimental.pallas import tpu as pltpu
from jax.experimental.pallas import tpu_sc as plsc

N = 10000
D = 128
E = 320000
NC = 2
NS = 16
NW = NC * NS
EPW = E // NW
CHUNK = 128
NFULL = EPW // CHUNK
TAIL = EPW - NFULL * CHUNK
RPT = N // NS
ZROWS = 25
WRows = 640
WLAST = N - 15 * WRows

_mesh = plsc.VectorSubcoreMesh(core_axis_name="c", subcore_axis_name="s")

_f32 = jnp.float32
_i32 = jnp.int32


@functools.partial(
    pl.kernel,
    mesh=_mesh,
    out_type=jax.ShapeDtypeStruct((NC * N, D), _f32),
    scratch_types=[
        pltpu.VMEM((CHUNK,), _i32),
        pltpu.VMEM((TAIL,), _i32),
        pltpu.VMEM((CHUNK, D), _f32),
        pltpu.VMEM((TAIL, D), _f32),
        pltpu.VMEM((ZROWS, D), _f32),
        pltpu.VMEM_SHARED((N, D), _f32),
    ],
)
def _deg_sc(row_hbm, out_hbm, rowv, rowt, onesb, onest, zbuf, acc):
    c = lax.axis_index("c")
    s = lax.axis_index("s")
    w = s * NC + c
    ones16 = jnp.ones((16,), _f32)
    zeros16 = jnp.zeros((16,), _f32)

    def fill(i, _):
        for j in range(D // 16):
            onesb[i, pl.ds(j * 16, 16)] = ones16
            zbuf[lax.rem(i, ZROWS), pl.ds(j * 16, 16)] = zeros16
        return 0
    lax.fori_loop(0, CHUNK, fill, 0)

    def fill_t(i, _):
        for j in range(D // 16):
            onest[i, pl.ds(j * 16, 16)] = ones16
        return 0
    lax.fori_loop(0, TAIL, fill_t, 0)

    def zero(i, _):
        pltpu.sync_copy(zbuf, acc.at[pl.ds(s * RPT + i * ZROWS, ZROWS)])
        return 0
    lax.fori_loop(0, RPT // ZROWS, zero, 0)
    plsc.subcore_barrier()

    base = w * EPW

    def step(j, _):
        pltpu.sync_copy(row_hbm.at[pl.ds(base + j * CHUNK, CHUNK)], rowv)
        pltpu.sync_copy(onesb, acc.at[rowv], add=True)
        return 0
    lax.fori_loop(0, NFULL, step, 0)
    pltpu.sync_copy(row_hbm.at[pl.ds(base + NFULL * CHUNK, TAIL)], rowt)
    pltpu.sync_copy(onest, acc.at[rowt], add=True)
    plsc.subcore_barrier()

    @pl.when(s < NS - 1)
    def _():
        pltpu.sync_copy(acc.at[pl.ds(s * WRows, WRows)],
                        out_hbm.at[pl.ds(c * N + s * WRows, WRows)])

    @pl.when(s == NS - 1)
    def _():
        pltpu.sync_copy(acc.at[pl.ds(15 * WRows, WLAST)],
                        out_hbm.at[pl.ds(c * N + 15 * WRows, WLAST)])


@functools.partial(
    pl.kernel,
    mesh=_mesh,
    out_type=jax.ShapeDtypeStruct((NC * N, D), _f32),
    scratch_types=[
        pltpu.VMEM((CHUNK,), _i32),
        pltpu.VMEM((CHUNK,), _i32),
        pltpu.VMEM((CHUNK,), _i32),
        pltpu.VMEM((CHUNK,), _i32),
        pltpu.VMEM((TAIL,), _i32),
        pltpu.VMEM((TAIL,), _i32),
        pltpu.VMEM((CHUNK, D), _f32),
        pltpu.VMEM((CHUNK, D), _f32),
        pltpu.VMEM((TAIL, D), _f32),
        pltpu.VMEM((ZROWS, D), _f32),
        pltpu.VMEM_SHARED((N, D), _f32),
        pltpu.SemaphoreType.DMA,
        pltpu.SemaphoreType.DMA,
        pltpu.SemaphoreType.DMA,
    ],
)
def _prop_sc(u_hbm, col_hbm, row_hbm, out_hbm,
             colv0, colv1, rowv0, rowv1, colt, rowt, b0, b1, buft, zbuf,
             acc, sem, ssem0, ssem1):
    c = lax.axis_index("c")
    s = lax.axis_index("s")
    w = s * NC + c
    zeros16 = jnp.zeros((16,), _f32)

    def zfill(i, _):
        for j in range(D // 16):
            zbuf[i, pl.ds(j * 16, 16)] = zeros16
        return 0
    lax.fori_loop(0, ZROWS, zfill, 0)

    def zero(i, _):
        pltpu.sync_copy(zbuf, acc.at[pl.ds(s * RPT + i * ZROWS, ZROWS)])
        return 0
    lax.fori_loop(0, RPT // ZROWS, zero, 0)
    plsc.subcore_barrier()

    base = w * EPW

    def _sdrain(ssem, buf):
        pltpu.make_async_copy(u_hbm.at[pl.ds(0, CHUNK)], buf, ssem).wait()

    def _gwait(buf):
        pltpu.make_async_copy(u_hbm.at[pl.ds(0, CHUNK)], buf, sem).wait()

    pltpu.sync_copy(col_hbm.at[pl.ds(base, CHUNK)], colv0)
    pltpu.sync_copy(row_hbm.at[pl.ds(base, CHUNK)], rowv0)
    pltpu.async_copy(u_hbm.at[colv0], b0, sem)

    def step(g, _):
        off = base + 2 * g * CHUNK
        pltpu.sync_copy(col_hbm.at[pl.ds(off + CHUNK, CHUNK)], colv1)
        pltpu.sync_copy(row_hbm.at[pl.ds(off + CHUNK, CHUNK)], rowv1)
        _gwait(b0)

        @pl.when(g > 0)
        def _():
            _sdrain(ssem1, b1)
        pltpu.async_copy(u_hbm.at[colv1], b1, sem)
        pltpu.async_copy(b0, acc.at[rowv0], ssem0, add=True)
        _gwait(b1)
        _sdrain(ssem0, b0)

        @pl.when(g < NFULL // 2 - 1)
        def _():
            offn = off + 2 * CHUNK
            pltpu.sync_copy(col_hbm.at[pl.ds(offn, CHUNK)], colv0)
            pltpu.sync_copy(row_hbm.at[pl.ds(offn, CHUNK)], rowv0)
            pltpu.async_copy(u_hbm.at[colv0], b0, sem)
        pltpu.async_copy(b1, acc.at[rowv1], ssem1, add=True)
        return 0
    lax.fori_loop(0, NFULL // 2, step, 0)
    _sdrain(ssem1, b1)

    offt = base + NFULL * CHUNK
    pltpu.sync_copy(col_hbm.at[pl.ds(offt, TAIL)], colt)
    pltpu.sync_copy(row_hbm.at[pl.ds(offt, TAIL)], rowt)
    pltpu.async_copy(u_hbm.at[colt], buft, sem).wait()
    pltpu.sync_copy(buft, acc.at[rowt], add=True)
    plsc.subcore_barrier()

    @pl.when(s < NS - 1)
    def _():
        pltpu.sync_copy(acc.at[pl.ds(s * WRows, WRows)],
                        out_hbm.at[pl.ds(c * N + s * WRows, WRows)])

    @pl.when(s == NS - 1)
    def _():
        pltpu.sync_copy(acc.at[pl.ds(15 * WRows, WLAST)],
                        out_hbm.at[pl.ds(c * N + 15 * WRows, WLAST)])


BM = 2000
GRID = N // BM

_sd = jax.ShapeDtypeStruct


def _dot(a, b):
    return jax.lax.dot_general(a, b, (((1,), (0,)), ((), ())),
                               precision=jax.lax.Precision.HIGHEST,
                               preferred_element_type=_f32)


def _stats_body(x_ref, m_ref, r_ref):
    x = x_ref[...]
    m = jnp.mean(x, axis=0, keepdims=True)
    m_ref[...] = m
    v = jnp.mean((x - m) * (x - m), axis=0, keepdims=True)
    r_ref[...] = jax.lax.rsqrt(v + 1e-5)


_bn_stats = pl.pallas_call(
    _stats_body,
    out_shape=(_sd((1, D), _f32), _sd((1, D), _f32)))


def _blk(shape):
    return pl.BlockSpec(shape, lambda i: (i, 0))


def _blk2(shape):
    return pl.BlockSpec(shape, lambda i: (i + GRID, 0))


def _full(shape):
    return pl.BlockSpec(shape, lambda i: (0, 0))


def _scale_body(x_ref, m_ref, r_ref, g_ref, be_ref, dga_ref, dgb_ref,
                xb_ref, dinv_ref, u1_ref):
    xb = ((x_ref[...] - m_ref[...]) * r_ref[...] * g_ref[...]
          + be_ref[...])
    xb_ref[...] = xb
    deg = dga_ref[:, 0:1] + dgb_ref[:, 0:1]
    dinv = jnp.where(deg > 0.0,
                     jax.lax.rsqrt(jnp.maximum(deg, 1.0)), 0.0)
    dinv_ref[...] = dinv
    u1_ref[...] = xb * dinv


_scale = pl.pallas_call(
    _scale_body,
    grid=(GRID,),
    in_specs=[_blk((BM, D)), _full((1, D)), _full((1, D)),
              _full((1, D)), _full((1, D)),
              _blk((BM, D)), _blk2((BM, D))],
    out_specs=(_blk((BM, D)), _blk((BM, 1)), _blk((BM, D))),
    out_shape=(_sd((N, D), _f32), _sd((N, 1), _f32), _sd((N, D), _f32)))


def _t3_body(sa_ref, sb_ref, dinv_ref, xb_ref, w0_ref, w1_ref, b_ref,
             g2_ref, acc1_ref):
    dinv = dinv_ref[...]
    tx1 = -dinv * (sa_ref[...] + sb_ref[...])
    g2_ref[...] = dinv * tx1
    acc1_ref[...] = (_dot(xb_ref[...], w0_ref[...])
                     + _dot(tx1, w1_ref[...]) + b_ref[...])


_t3 = pl.pallas_call(
    _t3_body,
    grid=(GRID,),
    in_specs=[_blk((BM, D)), _blk2((BM, D)), _blk((BM, 1)), _blk((BM, D)),
              _full((D, D)), _full((D, D)), _full((1, D))],
    out_specs=(_blk((BM, D)), _blk((BM, D))),
    out_shape=(_sd((N, D), _f32), _sd((N, D), _f32)))


def _t4a_body(sa_ref, sb_ref, dinv_ref, xb_ref, acc1_ref, w2_ref,
              h_ref):
    tx2 = (-2.0 * dinv_ref[...] * (sa_ref[...] + sb_ref[...])
           - xb_ref[...])
    h_ref[...] = jnp.maximum(acc1_ref[...] + _dot(tx2, w2_ref[...]), 0.0)


_t4a = pl.pallas_call(
    _t4a_body,
    grid=(GRID,),
    in_specs=[_blk((BM, D)), _blk2((BM, D)), _blk((BM, 1)), _blk((BM, D)),
              _blk((BM, D)), _full((D, D))],
    out_specs=_blk((BM, D)),
    out_shape=_sd((N, D), _f32))


def _t4b_body(h_ref, m_ref, r_ref, g_ref, be_ref, dinv_ref,
              hb_ref, g3_ref):
    hb = ((h_ref[...] - m_ref[...]) * r_ref[...] * g_ref[...]
          + be_ref[...])
    hb_ref[...] = hb
    g3_ref[...] = dinv_ref[...] * hb


_t4b = pl.pallas_call(
    _t4b_body,
    grid=(GRID,),
    in_specs=[_blk((BM, D)), _full((1, D)), _full((1, D)),
              _full((1, D)), _full((1, D)), _blk((BM, 1))],
    out_specs=(_blk((BM, D)), _blk((BM, D))),
    out_shape=(_sd((N, D), _f32), _sd((N, D), _f32)))


def _t5_body(sa_ref, sb_ref, dinv_ref, hb_ref, w0_ref, w1_ref, b_ref,
             acc2_ref, g4_ref):
    dinv = dinv_ref[...]
    ty1 = -dinv * (sa_ref[...] + sb_ref[...])
    g4_ref[...] = dinv * ty1
    acc2_ref[...] = (_dot(hb_ref[...], w0_ref[...])
                     + _dot(ty1, w1_ref[...]) + b_ref[...])


_t5 = pl.pallas_call(
    _t5_body,
    grid=(GRID,),
    in_specs=[_blk((BM, D)), _blk2((BM, D)), _blk((BM, 1)), _blk((BM, D)),
              _full((D, D)), _full((D, D)), _full((1, D))],
    out_specs=(_blk((BM, D)), _blk((BM, D))),
    out_shape=(_sd((N, D), _f32), _sd((N, D), _f32)))


def _t6_body(sa_ref, sb_ref, dinv_ref, hb_ref, acc2_ref, xb_ref, w2_ref,
             o_ref):
    ty2 = (-2.0 * dinv_ref[...] * (sa_ref[...] + sb_ref[...])
           - hb_ref[...])
    o_ref[...] = jnp.maximum(
        xb_ref[...] + acc2_ref[...] + _dot(ty2, w2_ref[...]), 0.0)


_t6 = pl.pallas_call(
    _t6_body,
    grid=(GRID,),
    in_specs=[_blk((BM, D)), _blk2((BM, D)), _blk((BM, 1)), _blk((BM, D)),
              _blk((BM, D)), _blk((BM, D)), _full((D, D))],
    out_specs=_blk((BM, D)),
    out_shape=_sd((N, D), _f32))


def kernel(x, edge_index, bn1_gamma, bn1_beta, W1, b1,
           bn2_gamma, bn2_beta, W2, b2):
    row = edge_index[0].astype(_i32)
    col = edge_index[1].astype(_i32)
    g1 = bn1_gamma.reshape(1, D)
    be1 = bn1_beta.reshape(1, D)
    g2r = bn2_gamma.reshape(1, D)
    be2 = bn2_beta.reshape(1, D)
    b1r = b1.reshape(1, D)
    b2r = b2.reshape(1, D)

    m1, r1 = _bn_stats(x)
    degp = _deg_sc(row)
    xb, dinv, u1 = _scale(x, m1, r1, g1, be1, degp, degp)
    s1 = _prop_sc(u1, col, row)
    g2, acc1 = _t3(s1, s1, dinv, xb, W1[0], W1[1], b1r)
    s2 = _prop_sc(g2, col, row)
    h = _t4a(s2, s2, dinv, xb, acc1, W1[2])
    m2, r2 = _bn_stats(h)
    hb, g3 = _t4b(h, m2, r2, g2r, be2, dinv)
    s3 = _prop_sc(g3, col, row)
    acc2, g4 = _t5(s3, s3, dinv, hb, W2[0], W2[1], b2r)
    s4 = _prop_sc(g4, col, row)
    return _t6(s4, s4, dinv, hb, acc2, xb, W2[2])
